# baseline (speedup 1.0000x reference)
import functools

import jax
import jax.numpy as jnp
from jax import lax
from jax.experimental import pallas as pl
from jax.experimental.pallas import tpu as pltpu
from jax.experimental.pallas import tpu_sc as plsc

LABS = 30
D = 128
B = 16384
NC = 2
NS = 16
NW = NC * NS
RPW = 2 * B // NW
CH = 128
NCH = RPW // CH

_mesh = plsc.VectorSubcoreMesh(core_axis_name="c", subcore_axis_name="s")


@functools.partial(
    pl.kernel,
    mesh=_mesh,
    out_type=jax.ShapeDtypeStruct((2 * B, D), jnp.float32),
    scratch_types=[
        pltpu.VMEM((NCH, CH), jnp.int32),
        pltpu.VMEM((2, CH, D), jnp.float32),
        pltpu.SemaphoreType.DMA,
        pltpu.SemaphoreType.DMA,
        pltpu.SemaphoreType.DMA,
        pltpu.SemaphoreType.DMA,
    ],
)
def _adapter(idx3, table, out, idxv, rows, gsem0, gsem1, wsem0, wsem1):
    wid = lax.axis_index("s") * NC + lax.axis_index("c")
    base = wid * RPW
    gsem = (gsem0, gsem1)
    wsem = (wsem0, wsem1)
    pltpu.sync_copy(idx3.at[wid], idxv)
    gets = {}
    puts = {}
    gets[0] = pltpu.async_copy(table.at[idxv.at[0]], rows.at[0], gsem[0])
    for j in range(NCH):
        b = j & 1
        gets[j].wait()
        if j + 1 < NCH:
            if j >= 1:
                puts[j - 1].wait()
            gets[j + 1] = pltpu.async_copy(
                table.at[idxv.at[j + 1]], rows.at[1 - b], gsem[1 - b])
        puts[j] = pltpu.async_copy(
            rows.at[b], out.at[pl.ds(base + j * CH, CH)], wsem[b])
    puts[NCH - 2].wait()
    puts[NCH - 1].wait()


def kernel(lab_ids, subject_ids, lab_table, subject_table):
    table = jnp.concatenate([lab_table, subject_table], axis=0)
    idx = jnp.stack([lab_ids, subject_ids + LABS], axis=1)
    out = _adapter(idx.reshape(NW, NCH, CH), table)
    return out.reshape(B, 2 * D)


# speedup vs baseline: 2.7999x; 2.7999x over previous
import functools

import jax
import jax.numpy as jnp
from jax import lax
from jax.experimental import pallas as pl
from jax.experimental.pallas import tpu as pltpu
from jax.experimental.pallas import tpu_sc as plsc

LABS = 30
SUBJ = 100
D = 128
D2 = 2 * D
B = 16384
NC = 2
NS = 16
NW = NC * NS
RPW = B // NW
CH = 128
NCH = RPW // CH

_mesh = plsc.VectorSubcoreMesh(core_axis_name="c", subcore_axis_name="s")


@functools.partial(
    pl.kernel,
    mesh=_mesh,
    out_type=jax.ShapeDtypeStruct((B, D2), jnp.float32),
    scratch_types=[
        pltpu.VMEM((NCH, CH), jnp.int32),
        pltpu.VMEM((2, CH, D2), jnp.float32),
        pltpu.SemaphoreType.DMA,
        pltpu.SemaphoreType.DMA,
        pltpu.SemaphoreType.DMA,
        pltpu.SemaphoreType.DMA,
    ],
)
def _adapter(idx3, table, out, idxv, rows, gsem0, gsem1, wsem0, wsem1):
    wid = lax.axis_index("s") * NC + lax.axis_index("c")
    base = wid * RPW
    gsem = (gsem0, gsem1)
    wsem = (wsem0, wsem1)
    pltpu.sync_copy(idx3.at[wid], idxv)
    gets = {}
    puts = {}
    gets[0] = pltpu.async_copy(table.at[idxv.at[0]], rows.at[0], gsem[0])
    for j in range(NCH):
        b = j & 1
        gets[j].wait()
        if j + 1 < NCH:
            if j >= 1:
                puts[j - 1].wait()
            gets[j + 1] = pltpu.async_copy(
                table.at[idxv.at[j + 1]], rows.at[1 - b], gsem[1 - b])
        puts[j] = pltpu.async_copy(
            rows.at[b], out.at[pl.ds(base + j * CH, CH)], wsem[b])
    puts[NCH - 2].wait()
    puts[NCH - 1].wait()


def kernel(lab_ids, subject_ids, lab_table, subject_table):
    paired = jnp.concatenate([
        jnp.broadcast_to(lab_table[:, None, :], (LABS, SUBJ, D)),
        jnp.broadcast_to(subject_table[None, :, :], (LABS, SUBJ, D)),
    ], axis=-1).reshape(LABS * SUBJ, D2)
    idx = lab_ids * SUBJ + subject_ids
    return _adapter(idx.reshape(NW, NCH, CH), paired)


# speedup vs baseline: 2.8451x; 1.0161x over previous
import functools

import jax
import jax.numpy as jnp
from jax import lax
from jax.experimental import pallas as pl
from jax.experimental.pallas import tpu as pltpu
from jax.experimental.pallas import tpu_sc as plsc

LABS = 30
SUBJ = 100
D = 128
D2 = 2 * D
B = 16384
NC = 2
NS = 16
NW = NC * NS
RPW = B // NW
CH = 64
NCH = RPW // CH
NBUF = 6

_mesh = plsc.VectorSubcoreMesh(core_axis_name="c", subcore_axis_name="s")


@functools.partial(
    pl.kernel,
    mesh=_mesh,
    out_type=jax.ShapeDtypeStruct((B, D2), jnp.float32),
    scratch_types=[
        pltpu.VMEM((RPW,), jnp.int32),
        pltpu.VMEM((NBUF, CH, D2), jnp.float32),
    ] + [pltpu.SemaphoreType.DMA] * 12,
)
def _adapter(idx1, table, out, idxv, rows, *sems):
    gsem = sems[:NBUF]
    wsem = sems[NBUF:]
    wid = lax.axis_index("s") * NC + lax.axis_index("c")
    base = wid * RPW
    pltpu.sync_copy(idx1.at[pl.ds(base, RPW)], idxv)
    gets = {}
    puts = {}
    for j in range(NBUF):
        gets[j] = pltpu.async_copy(
            table.at[idxv.at[pl.ds(j * CH, CH)]], rows.at[j], gsem[j])
    for j in range(NCH):
        b = j % NBUF
        if j >= NBUF:
            puts[j - NBUF].wait()
            gets[j] = pltpu.async_copy(
                table.at[idxv.at[pl.ds(j * CH, CH)]], rows.at[b], gsem[b])
        gets[j].wait()
        puts[j] = pltpu.async_copy(
            rows.at[b], out.at[pl.ds(base + j * CH, CH)], wsem[b])
    for j in range(NCH - NBUF, NCH):
        puts[j].wait()


def kernel(lab_ids, subject_ids, lab_table, subject_table):
    paired = jnp.concatenate([
        jnp.broadcast_to(lab_table[:, None, :], (LABS, SUBJ, D)),
        jnp.broadcast_to(subject_table[None, :, :], (LABS, SUBJ, D)),
    ], axis=-1).reshape(LABS * SUBJ, D2)
    idx = lab_ids * SUBJ + subject_ids
    return _adapter(idx, paired)
